# baseline (speedup 1.0000x reference)
import functools

import jax
import jax.numpy as jnp
from jax import lax
from jax.experimental import pallas as pl
from jax.experimental.pallas import tpu as pltpu
from jax.experimental.pallas import tpu_sc as plsc

NC = 2
NS = 16
NW = NC * NS

_MESH = functools.partial(
    plsc.VectorSubcoreMesh, core_axis_name="c", subcore_axis_name="s"
)


def _deg_kernel(E, NPAD, K, NCHUNK):
    PT = NPAD // NS
    G = 10
    assert NCHUNK % G == 0
    NGRP = NCHUNK // G

    @functools.partial(
        pl.kernel,
        out_type=jax.ShapeDtypeStruct((NC * NPAD,), jnp.float32),
        mesh=_MESH(),
        scratch_types=[
            pltpu.VMEM((PT,), jnp.float32),
            pltpu.VMEM((128,), jnp.float32),
            pltpu.VMEM((NCHUNK, K), jnp.int32),
            pltpu.VMEM_SHARED((NPAD,), jnp.float32),
            pltpu.SemaphoreType.DMA,
        ],
    )
    def deg_kernel(dst3_hbm, deg_out, zbuf, ones_v, idx_v, hist, sem):
        c = lax.axis_index("c")
        s = lax.axis_index("s")
        w = c * NS + s

        def fill_z(i, _):
            zbuf[pl.ds(i * 16, 16)] = jnp.zeros((16,), jnp.float32)
            return 0

        lax.fori_loop(0, PT // 16, fill_z, 0)
        for j in range(8):
            ones_v[pl.ds(j * 16, 16)] = jnp.ones((16,), jnp.float32)

        pltpu.sync_copy(dst3_hbm.at[w], idx_v)
        pltpu.sync_copy(zbuf, hist.at[pl.ds(s * PT, PT)])
        plsc.subcore_barrier()

        ones_k = ones_v.at[pl.ds(0, K)]

        def body(gi, _):
            for b in range(G):
                pltpu.async_copy(
                    ones_k, hist.at[idx_v.at[gi * G + b]], sem, add=True
                )
            for b in range(G):
                pltpu.make_async_copy(
                    ones_k, hist.at[idx_v.at[gi * G + b]], sem
                ).wait()
            return 0

        lax.fori_loop(0, NGRP, body, 0)
        plsc.subcore_barrier()
        pltpu.sync_copy(
            hist.at[pl.ds(s * PT, PT)],
            deg_out.at[pl.ds(c * NPAD + s * PT, PT)],
        )

    return deg_kernel


def _agg_kernel(N, NPAD, E, D, K, NCHUNK):
    RPT = N // NS
    CNT = ((RPT + 7) // 8) * 8
    NZ, ZREM = RPT // K, RPT % K
    NBUF = 2
    NPH = 2
    HCH = NCHUNK // NPH
    NGRP = HCH // NBUF

    EPT = E // NW

    @functools.partial(
        pl.kernel,
        out_type=jax.ShapeDtypeStruct((NC * NPAD, D), jnp.float32),
        mesh=_MESH(),
        scratch_types=[
            pltpu.VMEM((HCH, K), jnp.int32),
            pltpu.VMEM((HCH, K), jnp.int32),
            pltpu.VMEM((K, D), jnp.float32),
            pltpu.VMEM((K, D), jnp.float32),
            pltpu.VMEM_SHARED((N, D), jnp.float32),
            pltpu.SemaphoreType.DMA,
            pltpu.SemaphoreType.DMA,
        ],
    )
    def agg_kernel(g_hbm, src3_hbm, dst3_hbm, s_out,
                   sidx, didx, r0, r1, acc, m0, m1):
        c = lax.axis_index("c")
        s = lax.axis_index("s")
        w = c * NS + s
        rows = (r0, r1)
        sems = (m0, m1)

        def fill_z(i, _):
            for j in range(D // 16):
                r0[i, pl.ds(j * 16, 16)] = jnp.zeros((16,), jnp.float32)
            return 0

        lax.fori_loop(0, K, fill_z, 0)

        row0 = s * RPT
        for k in range(NZ):
            pltpu.sync_copy(r0, acc.at[pl.ds(row0 + k * K, K)])
        if ZREM:
            pltpu.sync_copy(
                r0.at[pl.ds(0, ZREM)], acc.at[pl.ds(row0 + NZ * K, ZREM)]
            )
        plsc.subcore_barrier()

        for ph in range(NPH):
            pltpu.sync_copy(src3_hbm.at[w, pl.ds(ph * HCH, HCH)], sidx)
            pltpu.sync_copy(dst3_hbm.at[w, pl.ds(ph * HCH, HCH)], didx)

            for b in range(NBUF):
                pltpu.async_copy(g_hbm.at[sidx.at[b]], rows[b], sems[b])

            def body(gi, _):
                for b in range(NBUF):
                    i = gi * NBUF + b
                    pltpu.make_async_copy(
                        g_hbm.at[sidx.at[i]], rows[b], sems[b]
                    ).wait()
                    pltpu.sync_copy(rows[b], acc.at[didx.at[i]], add=True)
                    pltpu.async_copy(
                        g_hbm.at[sidx.at[i + NBUF]], rows[b], sems[b]
                    )
                return 0

            lax.fori_loop(0, NGRP - 1, body, 0)
            for b in range(NBUF):
                i = HCH - NBUF + b
                pltpu.make_async_copy(
                    g_hbm.at[sidx.at[i]], rows[b], sems[b]
                ).wait()
                pltpu.sync_copy(rows[b], acc.at[didx.at[i]], add=True)

        plsc.subcore_barrier()

        bs = pl.multiple_of((s * RPT // 8) * 8, 8)
        bs = jnp.minimum(bs, N - CNT)
        pltpu.sync_copy(
            acc.at[pl.ds(bs, CNT)], s_out.at[pl.ds(c * NPAD + bs, CNT)]
        )

    return agg_kernel


def _dinv_col(deg_ref):
    d = deg_ref[0:1, :] + deg_ref[1:2, :] + 1.0
    return jnp.transpose(lax.rsqrt(d), (1, 0))


def _g_body(x_ref, w_ref, deg_ref, g_ref):
    h = lax.dot_general(
        x_ref[...], w_ref[...], (((1,), (1,)), ((), ())),
        preferred_element_type=jnp.float32,
    )
    g_ref[...] = _dinv_col(deg_ref) * h


def _fin_body(s0_ref, s1_ref, g_ref, deg_ref, b_ref, a_ref, o_ref):
    t = s0_ref[...] + s1_ref[...] + g_ref[...]
    t = _dinv_col(deg_ref) * t + b_ref[...]
    o_ref[...] = jnp.where(t >= 0.0, t, a_ref[0, 0] * t)


def kernel(x, edge_index, W, b, a):
    N, D = x.shape
    E = edge_index.shape[1]
    K = 125
    EPT = E // NW
    NCHUNK = EPT // K
    NPAD = ((N + NS * 16 - 1) // (NS * 16)) * (NS * 16)

    e4 = edge_index.reshape(2, NW, NCHUNK, K)
    src3 = e4[0]
    dst3 = e4[1]

    deg_flat = _deg_kernel(E, NPAD, K, NCHUNK)(dst3)
    deg2 = deg_flat.reshape(NC, NPAD)

    RB = 2048
    nb = NPAD // RB
    g = pl.pallas_call(
        _g_body,
        grid=(nb,),
        in_specs=[
            pl.BlockSpec((RB, D), lambda i: (i, 0)),
            pl.BlockSpec((D, D), lambda i: (0, 0)),
            pl.BlockSpec((NC, RB), lambda i: (0, i)),
        ],
        out_specs=pl.BlockSpec((RB, D), lambda i: (i, 0)),
        out_shape=jax.ShapeDtypeStruct((NPAD, D), jnp.float32),
    )(x, W, deg2)

    s_flat = _agg_kernel(N, NPAD, E, D, K, NCHUNK)(g, src3, dst3)

    out = pl.pallas_call(
        _fin_body,
        grid=(nb,),
        in_specs=[
            pl.BlockSpec((RB, D), lambda i: (i, 0)),
            pl.BlockSpec((RB, D), lambda i, nb=nb: (i + nb, 0)),
            pl.BlockSpec((RB, D), lambda i: (i, 0)),
            pl.BlockSpec((NC, RB), lambda i: (0, i)),
            pl.BlockSpec((1, D), lambda i: (0, 0)),
            pl.BlockSpec((1, 1), lambda i: (0, 0)),
        ],
        out_specs=pl.BlockSpec((RB, D), lambda i: (i, 0)),
        out_shape=jax.ShapeDtypeStruct((N, D), jnp.float32),
    )(s_flat, s_flat, g, deg2, jnp.reshape(b, (1, D)), jnp.reshape(a, (1, 1)))
    return out


# speedup vs baseline: 1.0181x; 1.0181x over previous
import functools

import jax
import jax.numpy as jnp
from jax import lax
from jax.experimental import pallas as pl
from jax.experimental.pallas import tpu as pltpu
from jax.experimental.pallas import tpu_sc as plsc

NC = 2
NS = 16
NW = NC * NS
CH = 128

_MESH = functools.partial(
    plsc.VectorSubcoreMesh, core_axis_name="c", subcore_axis_name="s"
)


def _deg_kernel(E, NPAD):
    PT = NPAD // NS
    TOTCH = E // CH
    JPT = ((TOTCH + NW - 1) // NW + 3) // 4 * 4

    @functools.partial(
        pl.kernel,
        out_type=jax.ShapeDtypeStruct((NC * NPAD,), jnp.float32),
        mesh=_MESH(),
        scratch_types=[
            pltpu.VMEM((PT,), jnp.float32),
            pltpu.VMEM((CH,), jnp.float32),
            pltpu.VMEM((2, CH), jnp.int32),
            pltpu.VMEM((2, CH), jnp.int32),
            pltpu.VMEM((2, CH), jnp.int32),
            pltpu.VMEM((2, CH), jnp.int32),
            pltpu.VMEM_SHARED((NPAD,), jnp.float32),
            pltpu.SemaphoreType.DMA,
            pltpu.SemaphoreType.DMA,
            pltpu.SemaphoreType.DMA,
            pltpu.SemaphoreType.DMA,
            pltpu.SemaphoreType.DMA,
            pltpu.SemaphoreType.DMA,
            pltpu.SemaphoreType.DMA,
            pltpu.SemaphoreType.DMA,
        ],
    )
    def deg_kernel(e_hbm, deg_out, zbuf, ones_v, i0, i1, i2, i3, hist,
                   m0, m1, m2, m3, s0, s1, s2, s3):
        c = lax.axis_index("c")
        s = lax.axis_index("s")
        w = c * NS + s
        ibufs = (i0, i1, i2, i3)
        isems = (m0, m1, m2, m3)
        ssems = (s0, s1, s2, s3)
        nvalid = (TOTCH - w + NW - 1) // NW

        def fill_z(i, _):
            zbuf[pl.ds(i * 16, 16)] = jnp.zeros((16,), jnp.float32)
            return 0

        lax.fori_loop(0, PT // 16, fill_z, 0)
        for j in range(CH // 16):
            ones_v[pl.ds(j * 16, 16)] = jnp.ones((16,), jnp.float32)

        def off(j):
            cid = jnp.minimum(j * NW + w, TOTCH - 1)
            return pl.multiple_of(cid * CH, CH)

        for q in range(2):
            pltpu.async_copy(e_hbm.at[:, pl.ds(off(q), CH)], ibufs[q], isems[q])

        pltpu.sync_copy(zbuf, hist.at[pl.ds(s * PT, PT)])
        plsc.subcore_barrier()

        def body(gi, _):
            for q in range(4):
                j = gi * 4 + q
                p = (q + 2) % 4
                pltpu.make_async_copy(
                    e_hbm.at[:, pl.ds(off(j), CH)], ibufs[q], isems[q]
                ).wait()

                @pl.when(j < nvalid)
                def _():
                    pltpu.async_copy(
                        ones_v, hist.at[ibufs[q].at[1]], ssems[q], add=True
                    )

                @pl.when(jnp.logical_and(j >= 2, j - 2 < nvalid))
                def _():
                    pltpu.make_async_copy(
                        ones_v, hist.at[ibufs[p].at[1]], ssems[p]
                    ).wait()

                @pl.when(j + 2 < JPT)
                def _():
                    pltpu.async_copy(
                        e_hbm.at[:, pl.ds(off(j + 2), CH)], ibufs[p], isems[p]
                    )
            return 0

        lax.fori_loop(0, JPT // 4, body, 0)

        for j in (JPT - 2, JPT - 1):
            @pl.when(j < nvalid)
            def _(j=j):
                pltpu.make_async_copy(
                    ones_v, hist.at[ibufs[j % 4].at[1]], ssems[j % 4]
                ).wait()
        plsc.subcore_barrier()
        pltpu.sync_copy(
            hist.at[pl.ds(s * PT, PT)],
            deg_out.at[pl.ds(c * NPAD + s * PT, PT)],
        )

    return deg_kernel


def _agg_kernel(N, NPAD, E, D):
    RPT = N // NS
    CNT = ((RPT + 7) // 8) * 8
    NZ, ZREM = RPT // CH, RPT % CH
    TOTCH = E // CH
    JPT = ((TOTCH + NW - 1) // NW + 3) // 4 * 4
    assert JPT % 4 == 0

    @functools.partial(
        pl.kernel,
        out_type=jax.ShapeDtypeStruct((NC * NPAD, D), jnp.float32),
        mesh=_MESH(),
        scratch_types=[
            pltpu.VMEM((2, CH), jnp.int32),
            pltpu.VMEM((2, CH), jnp.int32),
            pltpu.VMEM((2, CH), jnp.int32),
            pltpu.VMEM((2, CH), jnp.int32),
            pltpu.VMEM((CH, D), jnp.float32),
            pltpu.VMEM((CH, D), jnp.float32),
            pltpu.VMEM_SHARED((N, D), jnp.float32),
            pltpu.SemaphoreType.DMA,
            pltpu.SemaphoreType.DMA,
            pltpu.SemaphoreType.DMA,
            pltpu.SemaphoreType.DMA,
            pltpu.SemaphoreType.DMA,
            pltpu.SemaphoreType.DMA,
        ],
    )
    def agg_kernel(g_hbm, e_hbm, s_out, i0, i1, i2, i3, r0, r1, acc,
                   im0, im1, im2, im3, gm0, gm1):
        c = lax.axis_index("c")
        s = lax.axis_index("s")
        w = c * NS + s
        ibufs = (i0, i1, i2, i3)
        isems = (im0, im1, im2, im3)
        rows = (r0, r1)
        gsems = (gm0, gm1)
        nvalid = (TOTCH - w + NW - 1) // NW

        def off(j):
            cid = jnp.minimum(j * NW + w, TOTCH - 1)
            return pl.multiple_of(cid * CH, CH)

        def idx_load(j, q):
            pltpu.async_copy(
                e_hbm.at[:, pl.ds(off(j), CH)], ibufs[q], isems[q]
            )

        def idx_wait(j, q):
            pltpu.make_async_copy(
                e_hbm.at[:, pl.ds(off(j), CH)], ibufs[q], isems[q]
            ).wait()

        def gather(j, q, rb):
            pltpu.async_copy(g_hbm.at[ibufs[q].at[0]], rows[rb], gsems[rb])

        def gather_wait(j, q, rb):
            pltpu.make_async_copy(
                g_hbm.at[ibufs[q].at[0]], rows[rb], gsems[rb]
            ).wait()

        def scatter(j, q, rb):
            @pl.when(j < nvalid)
            def _():
                pltpu.sync_copy(rows[rb], acc.at[ibufs[q].at[1]], add=True)

        for q in range(4):
            idx_load(q, q)

        def fill_z(i, _):
            for jj in range(D // 16):
                r0[i, pl.ds(jj * 16, 16)] = jnp.zeros((16,), jnp.float32)
            return 0

        lax.fori_loop(0, CH, fill_z, 0)
        row0 = s * RPT
        for k in range(NZ):
            pltpu.sync_copy(r0, acc.at[pl.ds(row0 + k * CH, CH)])
        if ZREM:
            pltpu.sync_copy(
                r0.at[pl.ds(0, ZREM)], acc.at[pl.ds(row0 + NZ * CH, ZREM)]
            )
        plsc.subcore_barrier()

        for q in range(2):
            idx_wait(q, q)
            gather(q, q, q)

        def body(gi, _):
            for q in range(4):
                j = gi * 4 + q
                rb = q % 2
                gather_wait(j, q, rb)
                scatter(j, q, rb)
                idx_load(j + 4, q)
                idx_wait(j + 2, (q + 2) % 4)
                gather(j + 2, (q + 2) % 4, rb)
            return 0

        lax.fori_loop(0, JPT // 4 - 1, body, 0)

        for q in range(4):
            j = JPT - 4 + q
            rb = q % 2
            gather_wait(j, q, rb)
            scatter(j, q, rb)
            if q < 2:
                idx_wait(j + 2, (q + 2) % 4)
                gather(j + 2, (q + 2) % 4, rb)

        plsc.subcore_barrier()

        bs = pl.multiple_of((s * RPT // 8) * 8, 8)
        bs = jnp.minimum(bs, N - CNT)
        pltpu.sync_copy(
            acc.at[pl.ds(bs, CNT)], s_out.at[pl.ds(c * NPAD + bs, CNT)]
        )

    return agg_kernel


def _dinv_col(deg_ref):
    d = deg_ref[0:1, :] + deg_ref[1:2, :] + 1.0
    return jnp.transpose(lax.rsqrt(d), (1, 0))


def _g_body(x_ref, w_ref, deg_ref, g_ref):
    h = lax.dot_general(
        x_ref[...], w_ref[...], (((1,), (1,)), ((), ())),
        preferred_element_type=jnp.float32,
    )
    g_ref[...] = _dinv_col(deg_ref) * h


def _fin_body(s0_ref, s1_ref, g_ref, deg_ref, b_ref, a_ref, o_ref):
    t = s0_ref[...] + s1_ref[...] + g_ref[...]
    t = _dinv_col(deg_ref) * t + b_ref[...]
    o_ref[...] = jnp.where(t >= 0.0, t, a_ref[0, 0] * t)


def kernel(x, edge_index, W, b, a):
    N, D = x.shape
    E = edge_index.shape[1]
    NPAD = ((N + NS * 16 - 1) // (NS * 16)) * (NS * 16)

    deg_flat = _deg_kernel(E, NPAD)(edge_index)
    deg2 = deg_flat.reshape(NC, NPAD)

    RB = 2048
    nb = NPAD // RB
    g = pl.pallas_call(
        _g_body,
        grid=(nb,),
        in_specs=[
            pl.BlockSpec((RB, D), lambda i: (i, 0)),
            pl.BlockSpec((D, D), lambda i: (0, 0)),
            pl.BlockSpec((NC, RB), lambda i: (0, i)),
        ],
        out_specs=pl.BlockSpec((RB, D), lambda i: (i, 0)),
        out_shape=jax.ShapeDtypeStruct((NPAD, D), jnp.float32),
    )(x, W, deg2)

    s_flat = _agg_kernel(N, NPAD, E, D)(g, edge_index)

    out = pl.pallas_call(
        _fin_body,
        grid=(nb,),
        in_specs=[
            pl.BlockSpec((RB, D), lambda i: (i, 0)),
            pl.BlockSpec((RB, D), lambda i, nb=nb: (i + nb, 0)),
            pl.BlockSpec((RB, D), lambda i: (i, 0)),
            pl.BlockSpec((NC, RB), lambda i: (0, i)),
            pl.BlockSpec((1, D), lambda i: (0, 0)),
            pl.BlockSpec((1, 1), lambda i: (0, 0)),
        ],
        out_specs=pl.BlockSpec((RB, D), lambda i: (i, 0)),
        out_shape=jax.ShapeDtypeStruct((N, D), jnp.float32),
    )(s_flat, s_flat, g, deg2, jnp.reshape(b, (1, D)), jnp.reshape(a, (1, 1)))
    return out
